# baseline (speedup 1.0000x reference)
import jax
import jax.numpy as jnp
from jax.experimental import pallas as pl
from jax.experimental.pallas import tpu as pltpu

_B, _N, _C = 8, 20000, 80
_T = 100
_K = 300
_CONF = 0.8
_NMS = 0.4
_NPAD = 20480
_CHUNK = 512
_NEG = float("-inf")


def _col2row(v, n):
    r = jax.lax.broadcasted_iota(jnp.int32, (n, n), 0)
    c = jax.lax.broadcasted_iota(jnp.int32, (n, n), 1)
    m = jnp.where(r == c, jnp.broadcast_to(v, (n, n)), 0.0)
    return jnp.sum(m, axis=0, keepdims=True)


def _row2col(v, n):
    r = jax.lax.broadcasted_iota(jnp.int32, (n, n), 0)
    c = jax.lax.broadcasted_iota(jnp.int32, (n, n), 1)
    m = jnp.where(r == c, jnp.broadcast_to(v, (n, n)), 0.0)
    return jnp.sum(m, axis=1, keepdims=True)


def _det_kernel(lg_ref, boxes_ref, labels_ref, scores_ref,
                masked_ref, det_ref, ts_ref, iou_ref, bm_ref):
    rr = jax.lax.broadcasted_iota(jnp.int32, (_CHUNK, _CHUNK), 0)
    cc = jax.lax.broadcasted_iota(jnp.int32, (_CHUNK, _CHUNK), 1)
    for ci in range(_NPAD // _CHUNK):
        lo = ci * _CHUNK
        x = lg_ref[lo:lo + _CHUNK, :]
        obj = x[:, 4:5]
        cmax = jnp.max(x[:, 5:5 + _C], axis=1, keepdims=True)
        s = obj * cmax
        s = jnp.where(s > _CONF, s, _NEG)
        srow = jnp.sum(jnp.where(rr == cc, jnp.broadcast_to(s, (_CHUNK, _CHUNK)),
                                 0.0), axis=0, keepdims=True)
        for k in range(_CHUNK // 128):
            sub = srow[:, k * 128:(k + 1) * 128]
            col = ci * (_CHUNK // 128) + k
            masked_ref[col:col + 1, :] = sub
            bm_ref[:, col:col + 1] = jnp.max(sub, axis=1, keepdims=True)

    _NB = _NPAD // 128
    lane_b = jax.lax.broadcasted_iota(jnp.int32, (1, _NB), 1)
    lane_c = jax.lax.broadcasted_iota(jnp.int32, (1, 128), 1)
    lane_k = jax.lax.broadcasted_iota(jnp.int32, (1, _K), 1)

    def sel_body(k, carry):
        bm = bm_ref[...]
        m = jnp.max(bm)
        c = jnp.min(jnp.where(bm == m, lane_b, _NB))
        chunk = masked_ref[pl.ds(c, 1), :]
        j = jnp.min(jnp.where(chunk == m, lane_c, 128))
        idx = c * 128 + j
        row = lg_ref[pl.ds(idx, 1), :]
        det_ref[pl.ds(k, 1), :] = row
        ts_ref[...] = jnp.where(lane_k == k, m, ts_ref[...])
        nchunk = jnp.where(lane_c == j, _NEG, chunk)
        masked_ref[pl.ds(c, 1), :] = nchunk
        bm_ref[...] = jnp.where(lane_b == c, jnp.max(nchunk), bm)
        return carry

    jax.lax.fori_loop(0, _K, sel_body, 0)

    det = det_ref[...]
    cx, cy = det[:, 0:1], det[:, 1:2]
    w, h = det[:, 2:3], det[:, 3:4]
    x1 = cx - w * 0.5
    y1 = cy - h * 0.5
    x2 = x1 + w
    y2 = y1 + h
    area = jnp.maximum(x2 - x1, 0.0) * jnp.maximum(y2 - y1, 0.0)
    x1r = _col2row(x1, _K)
    y1r = _col2row(y1, _K)
    x2r = _col2row(x2, _K)
    y2r = _col2row(y2, _K)
    arear = _col2row(area, _K)
    iw = jnp.maximum(jnp.minimum(x2, x2r) - jnp.maximum(x1, x1r), 0.0)
    ih = jnp.maximum(jnp.minimum(y2, y2r) - jnp.maximum(y1, y1r), 0.0)
    inter = iw * ih
    union = area + arear - inter
    iou_ref[...] = inter / (union + 1e-9)

    valid = jnp.where(ts_ref[...] > -3e38, 1.0, 0.0)

    def nms_body(i, keep):
        row = iou_ref[pl.ds(i, 1), :]
        sup = jnp.max(jnp.where((row > _NMS) & (lane_k < i), keep, 0.0))
        return jnp.where((lane_k == i) & (sup > 0.0), 0.0, keep)

    keep = jax.lax.fori_loop(0, _K, nms_body, valid)

    ts = ts_ref[...]
    scores_ref[...] = jnp.where(keep > 0.0, ts, 0.0)
    kcol = _row2col(keep, _K)
    boxes_ref[...] = jnp.concatenate([x1, y1, x2, y2], axis=1) * kcol
    clsz = det[:, 5:5 + _C] * kcol
    m2 = jnp.max(clsz, axis=1, keepdims=True)
    colc = jax.lax.broadcasted_iota(jnp.int32, (_K, _C), 1)
    labels_ref[...] = jnp.min(jnp.where(clsz == m2, colc, _C),
                              axis=1, keepdims=True)


def _tgt_kernel(tg_ref, mk_ref, tb_ref, tl_ref, tsc_ref):
    t = tg_ref[...] * mk_ref[...]
    cx, cy = t[:, 0:1], t[:, 1:2]
    w, h = t[:, 2:3], t[:, 3:4]
    x1 = cx - w * 0.5
    y1 = cy - h * 0.5
    tb_ref[...] = jnp.concatenate([x1, y1, x1 + w, y1 + h], axis=1)
    tl_ref[...] = t[:, 5:6].astype(jnp.int32)
    tsc_ref[...] = t[:, 4:5]


def kernel(logits, targets, target_lengths):
    f32 = jnp.float32
    lp = jnp.pad(logits, ((0, 0), (0, _NPAD - _N), (0, 0)))

    boxes, labels, scores = pl.pallas_call(
        _det_kernel,
        grid=(_B,),
        in_specs=[
            pl.BlockSpec((None, _NPAD, 5 + _C), lambda b: (b, 0, 0)),
        ],
        out_specs=[
            pl.BlockSpec((None, _K, 4), lambda b: (b, 0, 0)),
            pl.BlockSpec((None, _K, 1), lambda b: (b, 0, 0)),
            pl.BlockSpec((None, 1, _K), lambda b: (b, 0, 0)),
        ],
        out_shape=[
            jax.ShapeDtypeStruct((_B, _K, 4), f32),
            jax.ShapeDtypeStruct((_B, _K, 1), jnp.int32),
            jax.ShapeDtypeStruct((_B, 1, _K), f32),
        ],
        scratch_shapes=[
            pltpu.VMEM((_NPAD // 128, 128), f32),
            pltpu.VMEM((_K, 5 + _C), f32),
            pltpu.VMEM((1, _K), f32),
            pltpu.VMEM((_K, _K), f32),
            pltpu.VMEM((1, _NPAD // 128), f32),
        ],
    )(lp)

    tmask = (jnp.arange(_T)[None, :] < target_lengths[:, None])
    tmask = tmask.astype(targets.dtype)[..., None]
    tb, tl, tsc = pl.pallas_call(
        _tgt_kernel,
        grid=(_B,),
        in_specs=[
            pl.BlockSpec((None, _T, 6), lambda b: (b, 0, 0)),
            pl.BlockSpec((None, _T, 1), lambda b: (b, 0, 0)),
        ],
        out_specs=[
            pl.BlockSpec((None, _T, 4), lambda b: (b, 0, 0)),
            pl.BlockSpec((None, _T, 1), lambda b: (b, 0, 0)),
            pl.BlockSpec((None, _T, 1), lambda b: (b, 0, 0)),
        ],
        out_shape=[
            jax.ShapeDtypeStruct((_B, _T, 4), f32),
            jax.ShapeDtypeStruct((_B, _T, 1), jnp.int32),
            jax.ShapeDtypeStruct((_B, _T, 1), f32),
        ],
    )(targets, tmask)

    return (boxes, labels[..., 0], scores[:, 0, :],
            tb, tl[..., 0], tsc[..., 0])


# speedup vs baseline: 1.1134x; 1.1134x over previous
import jax
import jax.numpy as jnp
from jax.experimental import pallas as pl
from jax.experimental.pallas import tpu as pltpu

_B, _N, _C = 8, 20000, 80
_T = 100
_K = 300
_CONF = 0.8
_NMS = 0.4
_NPAD = 20480
_CHUNK = 512
_NEG = float("-inf")


def _col2row(v, n):
    r = jax.lax.broadcasted_iota(jnp.int32, (n, n), 0)
    c = jax.lax.broadcasted_iota(jnp.int32, (n, n), 1)
    m = jnp.where(r == c, jnp.broadcast_to(v, (n, n)), 0.0)
    return jnp.sum(m, axis=0, keepdims=True)


def _row2col(v, n):
    r = jax.lax.broadcasted_iota(jnp.int32, (n, n), 0)
    c = jax.lax.broadcasted_iota(jnp.int32, (n, n), 1)
    m = jnp.where(r == c, jnp.broadcast_to(v, (n, n)), 0.0)
    return jnp.sum(m, axis=1, keepdims=True)


def _det_kernel(lg_ref, boxes_ref, labels_ref, scores_ref,
                masked_ref, det_ref, ts_ref, iou_ref):
    rr = jax.lax.broadcasted_iota(jnp.int32, (_CHUNK, _CHUNK), 0)
    cc = jax.lax.broadcasted_iota(jnp.int32, (_CHUNK, _CHUNK), 1)
    for ci in range(_NPAD // _CHUNK):
        lo = ci * _CHUNK
        x = lg_ref[lo:lo + _CHUNK, :]
        obj = x[:, 4:5]
        cmax = jnp.max(x[:, 5:5 + _C], axis=1, keepdims=True)
        s = obj * cmax
        s = jnp.where(s > _CONF, s, _NEG)
        srow = jnp.sum(jnp.where(rr == cc, jnp.broadcast_to(s, (_CHUNK, _CHUNK)),
                                 0.0), axis=0, keepdims=True)
        masked_ref[:, lo:lo + _CHUNK] = srow

    lane_n = jax.lax.broadcasted_iota(jnp.int32, (1, _NPAD), 1)
    lane_k = jax.lax.broadcasted_iota(jnp.int32, (1, _K), 1)

    def sel_body(k, carry):
        msk = masked_ref[...]
        m = jnp.max(msk)
        idx = jnp.min(jnp.where(msk == m, lane_n, _NPAD))
        row = lg_ref[pl.ds(idx, 1), :]
        det_ref[pl.ds(k, 1), :] = row
        ts_ref[...] = jnp.where(lane_k == k, m, ts_ref[...])
        masked_ref[...] = jnp.where(lane_n == idx, _NEG, msk)
        return carry

    jax.lax.fori_loop(0, _K, sel_body, 0)

    det = det_ref[...]
    cx, cy = det[:, 0:1], det[:, 1:2]
    w, h = det[:, 2:3], det[:, 3:4]
    x1 = cx - w * 0.5
    y1 = cy - h * 0.5
    x2 = x1 + w
    y2 = y1 + h
    area = jnp.maximum(x2 - x1, 0.0) * jnp.maximum(y2 - y1, 0.0)
    x1r = _col2row(x1, _K)
    y1r = _col2row(y1, _K)
    x2r = _col2row(x2, _K)
    y2r = _col2row(y2, _K)
    arear = _col2row(area, _K)
    iw = jnp.maximum(jnp.minimum(x2, x2r) - jnp.maximum(x1, x1r), 0.0)
    ih = jnp.maximum(jnp.minimum(y2, y2r) - jnp.maximum(y1, y1r), 0.0)
    inter = iw * ih
    union = area + arear - inter
    iou_ref[...] = inter / (union + 1e-9)

    valid = jnp.where(ts_ref[...] > -3e38, 1.0, 0.0)

    def nms_body(i, keep):
        row = iou_ref[pl.ds(i, 1), :]
        sup = jnp.max(jnp.where((row > _NMS) & (lane_k < i), keep, 0.0))
        return jnp.where((lane_k == i) & (sup > 0.0), 0.0, keep)

    keep = jax.lax.fori_loop(0, _K, nms_body, valid)

    ts = ts_ref[...]
    scores_ref[...] = jnp.where(keep > 0.0, ts, 0.0)
    kcol = _row2col(keep, _K)
    boxes_ref[...] = jnp.concatenate([x1, y1, x2, y2], axis=1) * kcol
    clsz = det[:, 5:5 + _C] * kcol
    m2 = jnp.max(clsz, axis=1, keepdims=True)
    colc = jax.lax.broadcasted_iota(jnp.int32, (_K, _C), 1)
    labels_ref[...] = jnp.min(jnp.where(clsz == m2, colc, _C),
                              axis=1, keepdims=True)


def _tgt_kernel(tg_ref, mk_ref, tb_ref, tl_ref, tsc_ref):
    t = tg_ref[...] * mk_ref[...]
    cx, cy = t[:, 0:1], t[:, 1:2]
    w, h = t[:, 2:3], t[:, 3:4]
    x1 = cx - w * 0.5
    y1 = cy - h * 0.5
    tb_ref[...] = jnp.concatenate([x1, y1, x1 + w, y1 + h], axis=1)
    tl_ref[...] = t[:, 5:6].astype(jnp.int32)
    tsc_ref[...] = t[:, 4:5]


def kernel(logits, targets, target_lengths):
    f32 = jnp.float32
    lp = jnp.pad(logits, ((0, 0), (0, _NPAD - _N), (0, 0)))

    boxes, labels, scores = pl.pallas_call(
        _det_kernel,
        grid=(_B,),
        in_specs=[
            pl.BlockSpec((None, _NPAD, 5 + _C), lambda b: (b, 0, 0)),
        ],
        out_specs=[
            pl.BlockSpec((None, _K, 4), lambda b: (b, 0, 0)),
            pl.BlockSpec((None, _K, 1), lambda b: (b, 0, 0)),
            pl.BlockSpec((None, 1, _K), lambda b: (b, 0, 0)),
        ],
        out_shape=[
            jax.ShapeDtypeStruct((_B, _K, 4), f32),
            jax.ShapeDtypeStruct((_B, _K, 1), jnp.int32),
            jax.ShapeDtypeStruct((_B, 1, _K), f32),
        ],
        scratch_shapes=[
            pltpu.VMEM((1, _NPAD), f32),
            pltpu.VMEM((_K, 5 + _C), f32),
            pltpu.VMEM((1, _K), f32),
            pltpu.VMEM((_K, _K), f32),
        ],
    )(lp)

    tmask = (jnp.arange(_T)[None, :] < target_lengths[:, None])
    tmask = tmask.astype(targets.dtype)[..., None]
    tb, tl, tsc = pl.pallas_call(
        _tgt_kernel,
        grid=(_B,),
        in_specs=[
            pl.BlockSpec((None, _T, 6), lambda b: (b, 0, 0)),
            pl.BlockSpec((None, _T, 1), lambda b: (b, 0, 0)),
        ],
        out_specs=[
            pl.BlockSpec((None, _T, 4), lambda b: (b, 0, 0)),
            pl.BlockSpec((None, _T, 1), lambda b: (b, 0, 0)),
            pl.BlockSpec((None, _T, 1), lambda b: (b, 0, 0)),
        ],
        out_shape=[
            jax.ShapeDtypeStruct((_B, _T, 4), f32),
            jax.ShapeDtypeStruct((_B, _T, 1), jnp.int32),
            jax.ShapeDtypeStruct((_B, _T, 1), f32),
        ],
    )(targets, tmask)

    return (boxes, labels[..., 0], scores[:, 0, :],
            tb, tl[..., 0], tsc[..., 0])
